# baseline (speedup 1.0000x reference)
import functools

import jax
import jax.numpy as jnp
import numpy as np
from jax.experimental import pallas as pl

B, S, D, H, G, DH = 1, 2048, 1024, 16, 4, 64
HG = H // G
L, STRIDE, LP, NSEL, W = 32, 16, 64, 8, 512
C = (S - L) // STRIDE + 1
CP = 128
NB = S // LP
SCALE = 1.0 / np.sqrt(DH)
NEG = -1e30

TS = 256
TQ = 256
KT = LP


def _proj_kernel(x_ref, wf_ref, cosi_ref, sins_ref, p_ref, q_ref, k_ref, v_ref):
    xw = jax.lax.dot_general(x_ref[...], wf_ref[...],
                             (((1,), (1,)), ((), ())),
                             preferred_element_type=jnp.float32)
    cosi = cosi_ref[...]
    sins = sins_ref[...]
    pm = p_ref[...]
    for h in range(H):
        u = xw[:, h * DH:(h + 1) * DH]
        q_ref[:, h * DH:(h + 1) * DH] = u * cosi + (u @ pm) * sins
    for g in range(G):
        u = xw[:, H * DH + g * DH: H * DH + (g + 1) * DH]
        k_ref[:, g * DH:(g + 1) * DH] = u * cosi + (u @ pm) * sins
    v_ref[...] = xw[:, (H + G) * DH:]


def _cmp_kernel(q_ref, k_ref, v_ref, wint_ref, ov_ref, cmp_ref, sel_ref):
    kc = jnp.dot(wint_ref[...], k_ref[0], preferred_element_type=jnp.float32)
    vc = jnp.dot(wint_ref[...], v_ref[0], preferred_element_type=jnp.float32)
    s_iota = jax.lax.broadcasted_iota(jnp.int32, (S, CP), 0)
    c_iota = jax.lax.broadcasted_iota(jnp.int32, (S, CP), 1)
    cmask = (STRIDE * c_iota + L - 1) <= s_iota
    imp = jnp.zeros((S, NB), jnp.float32)
    for h in range(HG):
        sc = jax.lax.dot_general(q_ref[0, h], kc, (((1,), (1,)), ((), ())),
                                 preferred_element_type=jnp.float32) * SCALE
        sc = jnp.where(cmask, sc, NEG)
        mx = jnp.max(sc, axis=1, keepdims=True)
        e = jnp.exp(sc - mx) * cmask.astype(jnp.float32)
        p = e / jnp.maximum(jnp.sum(e, axis=1, keepdims=True), 1e-30)
        cmp_ref[0, h] = jnp.dot(p, vc, preferred_element_type=jnp.float32)
        imp = imp + jnp.dot(p, ov_ref[...], preferred_element_type=jnp.float32)
    sj = jax.lax.broadcasted_iota(jnp.int32, (S, NB), 0)
    bj = jax.lax.broadcasted_iota(jnp.int32, (S, NB), 1)
    imp = imp + 1e9 * (bj == sj // LP) + 1e9 * (bj == 0)
    selected = jnp.zeros((S, NB), jnp.float32)
    for _ in range(NSEL):
        mx = jnp.max(imp, axis=1, keepdims=True)
        cand = jnp.where(imp == mx, bj, NB + 1)
        pick = bj == jnp.min(cand, axis=1, keepdims=True)
        selected = jnp.where(pick, 1.0, selected)
        imp = jnp.where(pick, -1.0, imp)
    sel_ref[0] = selected


def _flash_kernel(q_ref, k_ref, v_ref, sel_ref, cmp_ref, wg_ref, out_ref):
    qt = pl.program_id(1)
    qs = qt * TQ
    qv = q_ref[0]
    bs = sel_ref[0]
    lane = jax.lax.broadcasted_iota(jnp.int32, (TQ, KT), 1)
    row = jax.lax.broadcasted_iota(jnp.int32, (TQ, KT), 0) + qs
    nb_lane = jax.lax.broadcasted_iota(jnp.int32, (TQ, NB), 1)

    def tile(kt, carry, do_win):
        (m_s, l_s, a_s, m_w, l_w, a_w) = carry
        kblk = k_ref[0, pl.ds(kt * KT, KT), :]
        vblk = v_ref[0, pl.ds(kt * KT, KT), :]
        sf = jax.lax.dot_general(qv, kblk, (((1,), (1,)), ((), ())),
                                 preferred_element_type=jnp.float32) * SCALE
        t = kt * KT + lane
        causal = t <= row
        sel_flag = jnp.max(jnp.where(nb_lane == kt, bs, 0.0),
                           axis=1, keepdims=True) > 0.5
        sm = causal & sel_flag
        smf = sm.astype(jnp.float32)
        mn = jnp.maximum(m_s, jnp.max(jnp.where(sm, sf, NEG), axis=1,
                                      keepdims=True))
        p = jnp.exp(sf - mn) * smf
        alpha = jnp.exp(m_s - mn)
        l_s = l_s * alpha + jnp.sum(p, axis=1, keepdims=True)
        a_s = a_s * alpha + jnp.dot(p, vblk, preferred_element_type=jnp.float32)
        m_s = mn
        if do_win:
            wm = causal & (t > row - W)
            wmf = wm.astype(jnp.float32)
            mnw = jnp.maximum(m_w, jnp.max(jnp.where(wm, sf, NEG), axis=1,
                                           keepdims=True))
            pw = jnp.exp(sf - mnw) * wmf
            aw = jnp.exp(m_w - mnw)
            l_w = l_w * aw + jnp.sum(pw, axis=1, keepdims=True)
            a_w = a_w * aw + jnp.dot(pw, vblk,
                                     preferred_element_type=jnp.float32)
            m_w = mnw
        return (m_s, l_s, a_s, m_w, l_w, a_w)

    carry = (jnp.full((TQ, 1), NEG), jnp.zeros((TQ, 1), jnp.float32),
             jnp.zeros((TQ, DH), jnp.float32),
             jnp.full((TQ, 1), NEG), jnp.zeros((TQ, 1), jnp.float32),
             jnp.zeros((TQ, DH), jnp.float32))

    nkt = (qs + TQ) // KT
    kt_lo = jnp.maximum(qs - (W - 1), 0) // KT

    def sel_only(kt, carry):
        need = jnp.max(jnp.where(nb_lane == kt, bs, 0.0)) > 0.5
        return jax.lax.cond(need, lambda c: tile(kt, c, False),
                            lambda c: c, carry)

    carry = jax.lax.fori_loop(0, kt_lo, sel_only, carry)
    carry = jax.lax.fori_loop(kt_lo, nkt,
                              lambda kt, c: tile(kt, c, True), carry)
    (_, l_s, a_s, _, l_w, a_w) = carry
    out_sel = a_s / jnp.maximum(l_s, 1e-30)
    out_win = a_w / jnp.maximum(l_w, 1e-30)

    gm = jax.nn.sigmoid(jnp.dot(qv, wg_ref[...],
                                preferred_element_type=jnp.float32))
    glane = jax.lax.broadcasted_iota(jnp.int32, (TQ, CP), 1)
    g0 = jnp.sum(jnp.where(glane == 0, gm, 0.0), axis=1, keepdims=True)
    g1 = jnp.sum(jnp.where(glane == 1, gm, 0.0), axis=1, keepdims=True)
    g2 = jnp.sum(jnp.where(glane == 2, gm, 0.0), axis=1, keepdims=True)
    out_ref[0] = g0 * cmp_ref[0] + g1 * out_sel + g2 * out_win


def _oproj_kernel(o_ref, wo_ref, out_ref):
    out_ref[...] = jax.lax.dot_general(o_ref[...], wo_ref[...],
                                       (((1,), (1,)), ((), ())),
                                       preferred_element_type=jnp.float32)


def kernel(x, start_pos, freqs_cis, Wq, Wk, Wv, Wo, Wg):
    x2 = x.reshape(S, D)
    wf = jnp.concatenate([Wq, Wk, Wv], axis=0)
    cos = freqs_cis[..., 0]
    sin = freqs_cis[..., 1]
    cosi = jnp.repeat(cos, 2, axis=1)
    sins = jnp.stack([-sin, sin], axis=-1).reshape(S, DH)
    pmat = jnp.zeros((DH, DH), jnp.float32)
    idx = np.arange(0, DH, 2)
    pmat = pmat.at[idx + 1, idx].set(1.0).at[idx, idx + 1].set(1.0)

    q2, k2, v2 = pl.pallas_call(
        _proj_kernel,
        grid=(S // TS,),
        in_specs=[
            pl.BlockSpec((TS, D), lambda i: (i, 0)),
            pl.BlockSpec((H * DH + 2 * G * DH, D), lambda i: (0, 0)),
            pl.BlockSpec((TS, DH), lambda i: (i, 0)),
            pl.BlockSpec((TS, DH), lambda i: (i, 0)),
            pl.BlockSpec((DH, DH), lambda i: (0, 0)),
        ],
        out_specs=[
            pl.BlockSpec((TS, H * DH), lambda i: (i, 0)),
            pl.BlockSpec((TS, G * DH), lambda i: (i, 0)),
            pl.BlockSpec((TS, G * DH), lambda i: (i, 0)),
        ],
        out_shape=[
            jax.ShapeDtypeStruct((S, H * DH), jnp.float32),
            jax.ShapeDtypeStruct((S, G * DH), jnp.float32),
            jax.ShapeDtypeStruct((S, G * DH), jnp.float32),
        ],
    )(x2, wf, cosi, sins, pmat)

    qh = q2.reshape(S, H, DH).transpose(1, 0, 2)
    qg = qh.reshape(G, HG, S, DH)
    kg = k2.reshape(S, G, DH).transpose(1, 0, 2)
    vg = v2.reshape(S, G, DH).transpose(1, 0, 2)

    cidx = np.arange(CP)
    tidx = np.arange(S)
    win_np = ((tidx[None, :] >= STRIDE * cidx[:, None])
              & (tidx[None, :] < STRIDE * cidx[:, None] + L)
              & (cidx[:, None] < C)).astype(np.float32) / L
    wint = jnp.asarray(win_np)
    bstart = np.arange(NB) * LP
    ov_np = ((STRIDE * cidx[:, None] <= bstart[None, :] + LP - 1)
             & (STRIDE * cidx[:, None] + L - 1 >= bstart[None, :])
             & (cidx[:, None] < C)).astype(np.float32)
    ov = jnp.asarray(ov_np)

    out_cmp, blk_sel = pl.pallas_call(
        _cmp_kernel,
        grid=(G,),
        in_specs=[
            pl.BlockSpec((1, HG, S, DH), lambda g: (g, 0, 0, 0)),
            pl.BlockSpec((1, S, DH), lambda g: (g, 0, 0)),
            pl.BlockSpec((1, S, DH), lambda g: (g, 0, 0)),
            pl.BlockSpec((CP, S), lambda g: (0, 0)),
            pl.BlockSpec((CP, NB), lambda g: (0, 0)),
        ],
        out_specs=[
            pl.BlockSpec((1, HG, S, DH), lambda g: (g, 0, 0, 0)),
            pl.BlockSpec((1, S, NB), lambda g: (g, 0, 0)),
        ],
        out_shape=[
            jax.ShapeDtypeStruct((G, HG, S, DH), jnp.float32),
            jax.ShapeDtypeStruct((G, S, NB), jnp.float32),
        ],
    )(qg, kg, vg, wint, ov)

    cmp_h = out_cmp.reshape(H, S, DH)
    wg_pad = jnp.zeros((DH, CP), jnp.float32).at[:, :3].set(Wg)

    out_h = pl.pallas_call(
        _flash_kernel,
        grid=(H, S // TQ),
        in_specs=[
            pl.BlockSpec((1, TQ, DH), lambda h, i: (h, i, 0)),
            pl.BlockSpec((1, S, DH), lambda h, i: (h // HG, 0, 0)),
            pl.BlockSpec((1, S, DH), lambda h, i: (h // HG, 0, 0)),
            pl.BlockSpec((1, TQ, NB), lambda h, i: (h // HG, i, 0)),
            pl.BlockSpec((1, TQ, DH), lambda h, i: (h, i, 0)),
            pl.BlockSpec((DH, CP), lambda h, i: (0, 0)),
        ],
        out_specs=pl.BlockSpec((1, TQ, DH), lambda h, i: (h, i, 0)),
        out_shape=jax.ShapeDtypeStruct((H, S, DH), jnp.float32),
    )(qh, kg, vg, blk_sel, cmp_h, wg_pad)

    o2 = out_h.transpose(1, 0, 2).reshape(S, H * DH)
    out = pl.pallas_call(
        _oproj_kernel,
        grid=(S // TS,),
        in_specs=[
            pl.BlockSpec((TS, H * DH), lambda i: (i, 0)),
            pl.BlockSpec((D, H * DH), lambda i: (0, 0)),
        ],
        out_specs=pl.BlockSpec((TS, D), lambda i: (i, 0)),
        out_shape=jax.ShapeDtypeStruct((S, D), jnp.float32),
    )(o2, Wo)
    return out.reshape(B, S, D)


# speedup vs baseline: 4.5200x; 4.5200x over previous
import jax
import jax.numpy as jnp
import numpy as np
from jax.experimental import pallas as pl
from jax.experimental.pallas import tpu as pltpu

B, S, D, H, G, DH = 1, 2048, 1024, 16, 4, 64
HG = H // G
L, STRIDE, LP, NSEL, W = 32, 16, 64, 8, 512
C = (S - L) // STRIDE + 1
CP = 128
NB = S // LP
SCALE = 1.0 / np.sqrt(DH)
NEG = -1e30

TS = 256
TQ = 256
KT = 256
BPT = KT // LP


def _proj_kernel(x_ref, wf_ref, cosi_ref, sins_ref, p_ref, q_ref, k_ref, v_ref):
    xw = jax.lax.dot_general(x_ref[...], wf_ref[...],
                             (((1,), (1,)), ((), ())),
                             preferred_element_type=jnp.float32)
    cosi = cosi_ref[...]
    sins = sins_ref[...]
    pm = p_ref[...]
    for h in range(H):
        u = xw[:, h * DH:(h + 1) * DH]
        q_ref[:, h * DH:(h + 1) * DH] = u * cosi + (u @ pm) * sins
    for g in range(G):
        u = xw[:, H * DH + g * DH: H * DH + (g + 1) * DH]
        k_ref[:, g * DH:(g + 1) * DH] = u * cosi + (u @ pm) * sins
    v_ref[...] = xw[:, (H + G) * DH:]


def _cmp_kernel(q_ref, k_ref, v_ref, wint_ref, ov_ref, cmp_ref, sel_ref):
    kc = jnp.dot(wint_ref[...], k_ref[0], preferred_element_type=jnp.float32)
    vc = jnp.dot(wint_ref[...], v_ref[0], preferred_element_type=jnp.float32)
    s_iota = jax.lax.broadcasted_iota(jnp.int32, (S, CP), 0)
    c_iota = jax.lax.broadcasted_iota(jnp.int32, (S, CP), 1)
    cmask = (STRIDE * c_iota + L - 1) <= s_iota
    imp = jnp.zeros((S, NB), jnp.float32)
    for h in range(HG):
        sc = jax.lax.dot_general(q_ref[0, h], kc, (((1,), (1,)), ((), ())),
                                 preferred_element_type=jnp.float32) * SCALE
        sc = jnp.where(cmask, sc, NEG)
        mx = jnp.max(sc, axis=1, keepdims=True)
        e = jnp.exp(sc - mx) * cmask.astype(jnp.float32)
        p = e / jnp.maximum(jnp.sum(e, axis=1, keepdims=True), 1e-30)
        cmp_ref[0, h] = jnp.dot(p, vc, preferred_element_type=jnp.float32)
        imp = imp + jnp.dot(p, ov_ref[...], preferred_element_type=jnp.float32)
    sj = jax.lax.broadcasted_iota(jnp.int32, (S, NB), 0)
    bj = jax.lax.broadcasted_iota(jnp.int32, (S, NB), 1)
    forced = (bj == sj // LP) | (bj == 0)
    quota = jnp.where(sj[:, :1] // LP == 0, NSEL - 1, NSEL - 2)
    selected = forced.astype(jnp.float32)
    work = jnp.where(forced, -1.0, imp)
    for i in range(NSEL - 1):
        mx = jnp.max(work, axis=1, keepdims=True)
        pick = (work == mx) & (i < quota)
        selected = jnp.where(pick, 1.0, selected)
        work = jnp.where(pick, -1.0, work)
    sel_ref[0] = selected


def _flash_kernel(q_ref, k_ref, v_ref, sel_ref, cmp_ref, wg_ref, e4_ref,
                  out_ref):
    lane = jax.lax.broadcasted_iota(jnp.int32, (TQ, KT), 1)
    row = jax.lax.broadcasted_iota(jnp.int32, (TQ, KT), 0)
    d = lane - row
    causal = d <= 0
    causal_f = causal.astype(jnp.float32)
    edge = d > 0
    edge_f = edge.astype(jnp.float32)
    glane = jax.lax.broadcasted_iota(jnp.int32, (TQ, CP), 1)
    e4 = e4_ref[...]

    for qt in range(S // TQ):
        qsl = slice(qt * TQ, (qt + 1) * TQ)
        qv = q_ref[0, qsl, :]
        bs = sel_ref[0, qsl, :]
        m_s = jnp.full((TQ, 1), NEG)
        l_s = jnp.zeros((TQ, 1), jnp.float32)
        a_s = jnp.zeros((TQ, DH), jnp.float32)
        m_w = jnp.full((TQ, 1), NEG)
        l_w = jnp.zeros((TQ, 1), jnp.float32)
        a_w = jnp.zeros((TQ, DH), jnp.float32)
        for kt in range(qt + 1):
            ksl = slice(kt * KT, (kt + 1) * KT)
            kblk = k_ref[0, ksl, :]
            vblk = v_ref[0, ksl, :]
            sf = jax.lax.dot_general(qv, kblk, (((1,), (1,)), ((), ())),
                                     preferred_element_type=jnp.float32)
            sf = sf * SCALE
            smf = jnp.dot(bs[:, kt * BPT:(kt + 1) * BPT], e4,
                          preferred_element_type=jnp.float32)
            if kt == qt:
                smf = smf * causal_f
            sm = smf > 0.5
            mn = jnp.maximum(m_s, jnp.max(jnp.where(sm, sf, NEG), axis=1,
                                          keepdims=True))
            p = jnp.exp(sf - mn) * smf
            alpha = jnp.exp(m_s - mn)
            l_s = l_s * alpha + jnp.sum(p, axis=1, keepdims=True)
            a_s = a_s * alpha + jnp.dot(p, vblk,
                                        preferred_element_type=jnp.float32)
            m_s = mn
            if kt >= qt - 2:
                if kt == qt:
                    wm, wmf = causal, causal_f
                elif kt == qt - 1:
                    wm, wmf = None, None
                else:
                    wm, wmf = edge, edge_f
                if wm is None:
                    mnw = jnp.maximum(m_w, jnp.max(sf, axis=1, keepdims=True))
                    pw = jnp.exp(sf - mnw)
                else:
                    mnw = jnp.maximum(m_w, jnp.max(jnp.where(wm, sf, NEG),
                                                   axis=1, keepdims=True))
                    pw = jnp.exp(sf - mnw) * wmf
                aw = jnp.exp(m_w - mnw)
                l_w = l_w * aw + jnp.sum(pw, axis=1, keepdims=True)
                a_w = a_w * aw + jnp.dot(pw, vblk,
                                         preferred_element_type=jnp.float32)
                m_w = mnw
        out_sel = a_s / jnp.maximum(l_s, 1e-30)
        out_win = a_w / jnp.maximum(l_w, 1e-30)
        gm = jax.nn.sigmoid(jnp.dot(qv, wg_ref[...],
                                    preferred_element_type=jnp.float32))
        g0 = jnp.sum(jnp.where(glane == 0, gm, 0.0), axis=1, keepdims=True)
        g1 = jnp.sum(jnp.where(glane == 1, gm, 0.0), axis=1, keepdims=True)
        g2 = jnp.sum(jnp.where(glane == 2, gm, 0.0), axis=1, keepdims=True)
        out_ref[0, qsl, :] = g0 * cmp_ref[0, qsl, :] + g1 * out_sel + g2 * out_win


def _oproj_kernel(o_ref, wo_ref, out_ref):
    out_ref[...] = jax.lax.dot_general(o_ref[...], wo_ref[...],
                                       (((1,), (1,)), ((), ())),
                                       preferred_element_type=jnp.float32)


def kernel(x, start_pos, freqs_cis, Wq, Wk, Wv, Wo, Wg):
    x2 = x.reshape(S, D)
    wf = jnp.concatenate([Wq, Wk, Wv], axis=0)
    cos = freqs_cis[..., 0]
    sin = freqs_cis[..., 1]
    cosi = jnp.repeat(cos, 2, axis=1)
    sins = jnp.stack([-sin, sin], axis=-1).reshape(S, DH)
    pmat = jnp.zeros((DH, DH), jnp.float32)
    idx = np.arange(0, DH, 2)
    pmat = pmat.at[idx + 1, idx].set(1.0).at[idx, idx + 1].set(1.0)

    q2, k2, v2 = pl.pallas_call(
        _proj_kernel,
        grid=(S // TS,),
        in_specs=[
            pl.BlockSpec((TS, D), lambda i: (i, 0)),
            pl.BlockSpec((H * DH + 2 * G * DH, D), lambda i: (0, 0)),
            pl.BlockSpec((TS, DH), lambda i: (i, 0)),
            pl.BlockSpec((TS, DH), lambda i: (i, 0)),
            pl.BlockSpec((DH, DH), lambda i: (0, 0)),
        ],
        out_specs=[
            pl.BlockSpec((TS, H * DH), lambda i: (i, 0)),
            pl.BlockSpec((TS, G * DH), lambda i: (i, 0)),
            pl.BlockSpec((TS, G * DH), lambda i: (i, 0)),
        ],
        out_shape=[
            jax.ShapeDtypeStruct((S, H * DH), jnp.float32),
            jax.ShapeDtypeStruct((S, G * DH), jnp.float32),
            jax.ShapeDtypeStruct((S, G * DH), jnp.float32),
        ],
        compiler_params=pltpu.CompilerParams(
            dimension_semantics=("parallel",)),
    )(x2, wf, cosi, sins, pmat)

    qh = q2.reshape(S, H, DH).transpose(1, 0, 2)
    qg = qh.reshape(G, HG, S, DH)
    kg = k2.reshape(S, G, DH).transpose(1, 0, 2)
    vg = v2.reshape(S, G, DH).transpose(1, 0, 2)

    cidx = np.arange(CP)
    tidx = np.arange(S)
    win_np = ((tidx[None, :] >= STRIDE * cidx[:, None])
              & (tidx[None, :] < STRIDE * cidx[:, None] + L)
              & (cidx[:, None] < C)).astype(np.float32) / L
    wint = jnp.asarray(win_np)
    bstart = np.arange(NB) * LP
    ov_np = ((STRIDE * cidx[:, None] <= bstart[None, :] + LP - 1)
             & (STRIDE * cidx[:, None] + L - 1 >= bstart[None, :])
             & (cidx[:, None] < C)).astype(np.float32)
    ov = jnp.asarray(ov_np)

    out_cmp, blk_sel = pl.pallas_call(
        _cmp_kernel,
        grid=(G,),
        in_specs=[
            pl.BlockSpec((1, HG, S, DH), lambda g: (g, 0, 0, 0)),
            pl.BlockSpec((1, S, DH), lambda g: (g, 0, 0)),
            pl.BlockSpec((1, S, DH), lambda g: (g, 0, 0)),
            pl.BlockSpec((CP, S), lambda g: (0, 0)),
            pl.BlockSpec((CP, NB), lambda g: (0, 0)),
        ],
        out_specs=[
            pl.BlockSpec((1, HG, S, DH), lambda g: (g, 0, 0, 0)),
            pl.BlockSpec((1, S, NB), lambda g: (g, 0, 0)),
        ],
        out_shape=[
            jax.ShapeDtypeStruct((G, HG, S, DH), jnp.float32),
            jax.ShapeDtypeStruct((G, S, NB), jnp.float32),
        ],
        compiler_params=pltpu.CompilerParams(
            dimension_semantics=("parallel",)),
    )(qg, kg, vg, wint, ov)

    cmp_h = out_cmp.reshape(H, S, DH)
    wg_pad = jnp.zeros((DH, CP), jnp.float32).at[:, :3].set(Wg)
    e4_np = (np.arange(KT)[None, :] // LP == np.arange(BPT)[:, None])
    e4 = jnp.asarray(e4_np.astype(np.float32))

    out_h = pl.pallas_call(
        _flash_kernel,
        grid=(H,),
        in_specs=[
            pl.BlockSpec((1, S, DH), lambda h: (h, 0, 0)),
            pl.BlockSpec((1, S, DH), lambda h: (h // HG, 0, 0)),
            pl.BlockSpec((1, S, DH), lambda h: (h // HG, 0, 0)),
            pl.BlockSpec((1, S, NB), lambda h: (h // HG, 0, 0)),
            pl.BlockSpec((1, S, DH), lambda h: (h, 0, 0)),
            pl.BlockSpec((DH, CP), lambda h: (0, 0)),
            pl.BlockSpec((BPT, KT), lambda h: (0, 0)),
        ],
        out_specs=pl.BlockSpec((1, S, DH), lambda h: (h, 0, 0)),
        out_shape=jax.ShapeDtypeStruct((H, S, DH), jnp.float32),
        compiler_params=pltpu.CompilerParams(
            dimension_semantics=("parallel",)),
    )(qh, kg, vg, blk_sel, cmp_h, wg_pad, e4)

    o2 = out_h.transpose(1, 0, 2).reshape(S, H * DH)
    out = pl.pallas_call(
        _oproj_kernel,
        grid=(S // TS,),
        in_specs=[
            pl.BlockSpec((TS, H * DH), lambda i: (i, 0)),
            pl.BlockSpec((D, H * DH), lambda i: (0, 0)),
        ],
        out_specs=pl.BlockSpec((TS, D), lambda i: (i, 0)),
        out_shape=jax.ShapeDtypeStruct((S, D), jnp.float32),
        compiler_params=pltpu.CompilerParams(
            dimension_semantics=("parallel",)),
    )(o2, Wo)
    return out.reshape(B, S, D)


# speedup vs baseline: 5.0112x; 1.1087x over previous
import jax
import jax.numpy as jnp
import numpy as np
from jax.experimental import pallas as pl
from jax.experimental.pallas import tpu as pltpu

B, S, D, H, G, DH = 1, 2048, 1024, 1024 // 64, 4, 64
HG = H // G
L, STRIDE, LP, NSEL, W = 32, 16, 64, 8, 512
C = (S - L) // STRIDE + 1
CP = 128
NB = S // LP
SCALE = 1.0 / np.sqrt(DH)
NEG = -1e30

TS = 256
TQ = 256
KT = 256
BPT = KT // LP
BF = jnp.bfloat16


def _proj_kernel(x_ref, wf_ref, cosi_ref, sins_ref, p_ref,
                 q_ref, k_ref, v_ref, qb_ref, kb_ref, vb_ref):
    xw = jax.lax.dot_general(x_ref[...], wf_ref[...],
                             (((1,), (1,)), ((), ())),
                             preferred_element_type=jnp.float32)
    cosi = cosi_ref[...]
    sins = sins_ref[...]
    pm = p_ref[...]
    for h in range(H):
        u = xw[:, h * DH:(h + 1) * DH]
        r = u * cosi + (u @ pm) * sins
        q_ref[h] = r
        qb_ref[h] = r.astype(BF)
    for g in range(G):
        u = xw[:, H * DH + g * DH: H * DH + (g + 1) * DH]
        r = u * cosi + (u @ pm) * sins
        k_ref[g] = r
        kb_ref[g] = (r * SCALE).astype(BF)
        w = xw[:, (H + G) * DH + g * DH: (H + G) * DH + (g + 1) * DH]
        v_ref[g] = w
        vb_ref[g] = w.astype(BF)


def _cmp_kernel(q_ref, k_ref, v_ref, wint_ref, ov_ref, cmp_ref, sel_ref):
    kc = jnp.dot(wint_ref[...], k_ref[0], preferred_element_type=jnp.float32)
    vc = jnp.dot(wint_ref[...], v_ref[0], preferred_element_type=jnp.float32)
    s_iota = jax.lax.broadcasted_iota(jnp.int32, (S, CP), 0)
    c_iota = jax.lax.broadcasted_iota(jnp.int32, (S, CP), 1)
    cmask = (STRIDE * c_iota + L - 1) <= s_iota
    cmask_f = cmask.astype(jnp.float32)
    imp = jnp.zeros((S, NB), jnp.float32)
    for h in range(HG):
        sc = jax.lax.dot_general(q_ref[0, h], kc, (((1,), (1,)), ((), ())),
                                 preferred_element_type=jnp.float32) * SCALE
        sc = jnp.where(cmask, sc, NEG)
        mx = jnp.max(sc, axis=1, keepdims=True)
        e = jnp.exp(sc - mx) * cmask_f
        p = e / jnp.maximum(jnp.sum(e, axis=1, keepdims=True), 1e-30)
        cmp_ref[0, h] = jnp.dot(p, vc, preferred_element_type=jnp.float32)
        imp = imp + jnp.dot(p, ov_ref[...], preferred_element_type=jnp.float32)
    sj = jax.lax.broadcasted_iota(jnp.int32, (S, NB), 0)
    bj = jax.lax.broadcasted_iota(jnp.int32, (S, NB), 1)
    forced = (bj == sj // LP) | (bj == 0)
    quota = jnp.where(sj[:, :1] // LP == 0, NSEL - 1, NSEL - 2)
    selected = forced.astype(jnp.float32)
    work = jnp.where(forced, -1.0, imp)
    for i in range(NSEL - 1):
        mx = jnp.max(work, axis=1, keepdims=True)
        pick = (work == mx) & (i < quota)
        selected = jnp.where(pick, 1.0, selected)
        work = jnp.where(pick, -1.0, work)
    sel_ref[0] = selected


def _flash_kernel(q_ref, k_ref, v_ref, sel_ref, cmp_ref, wg_ref, e4_ref,
                  out_ref):
    lane = jax.lax.broadcasted_iota(jnp.int32, (TQ, KT), 1)
    row = jax.lax.broadcasted_iota(jnp.int32, (TQ, KT), 0)
    d = lane - row
    causal = d <= 0
    causal_f = causal.astype(jnp.float32)
    edge = d > 0
    edge_f = edge.astype(jnp.float32)
    glane = jax.lax.broadcasted_iota(jnp.int32, (TQ, CP), 1)
    e4 = e4_ref[...]

    for qt in range(S // TQ):
        qsl = slice(qt * TQ, (qt + 1) * TQ)
        qv = q_ref[0, qsl, :]
        bs = sel_ref[0, qsl, :]
        m_s = jnp.full((TQ, 1), NEG)
        l_s = jnp.zeros((TQ, 1), jnp.float32)
        a_s = jnp.zeros((TQ, DH), jnp.float32)
        m_w = jnp.full((TQ, 1), NEG)
        l_w = jnp.zeros((TQ, 1), jnp.float32)
        a_w = jnp.zeros((TQ, DH), jnp.float32)
        for kt in range(qt + 1):
            ksl = slice(kt * KT, (kt + 1) * KT)
            kblk = k_ref[0, ksl, :]
            vblk = v_ref[0, ksl, :]
            sf = jax.lax.dot_general(qv, kblk, (((1,), (1,)), ((), ())),
                                     preferred_element_type=jnp.float32)
            smf = jnp.dot(bs[:, kt * BPT:(kt + 1) * BPT], e4,
                          preferred_element_type=jnp.float32)
            if kt == qt:
                smf = smf * causal_f
            sm = smf > 0.5
            mn = jnp.maximum(m_s, jnp.max(jnp.where(sm, sf, NEG), axis=1,
                                          keepdims=True))
            p = jnp.exp(sf - mn) * smf
            alpha = jnp.exp(m_s - mn)
            l_s = l_s * alpha + jnp.sum(p, axis=1, keepdims=True)
            a_s = a_s * alpha + jnp.dot(p.astype(BF), vblk,
                                        preferred_element_type=jnp.float32)
            m_s = mn
            if kt >= qt - 2:
                if kt == qt:
                    wm, wmf = causal, causal_f
                elif kt == qt - 1:
                    wm, wmf = None, None
                else:
                    wm, wmf = edge, edge_f
                if wm is None:
                    mnw = jnp.maximum(m_w, jnp.max(sf, axis=1, keepdims=True))
                    pw = jnp.exp(sf - mnw)
                else:
                    mnw = jnp.maximum(m_w, jnp.max(jnp.where(wm, sf, NEG),
                                                   axis=1, keepdims=True))
                    pw = jnp.exp(sf - mnw) * wmf
                aw = jnp.exp(m_w - mnw)
                l_w = l_w * aw + jnp.sum(pw, axis=1, keepdims=True)
                a_w = a_w * aw + jnp.dot(pw.astype(BF), vblk,
                                         preferred_element_type=jnp.float32)
                m_w = mnw
        out_sel = a_s / jnp.maximum(l_s, 1e-30)
        out_win = a_w / jnp.maximum(l_w, 1e-30)
        gm = jax.nn.sigmoid(jax.lax.dot_general(
            qv, wg_ref[...], (((1,), (0,)), ((), ())),
            preferred_element_type=jnp.float32))
        g0 = jnp.sum(jnp.where(glane == 0, gm, 0.0), axis=1, keepdims=True)
        g1 = jnp.sum(jnp.where(glane == 1, gm, 0.0), axis=1, keepdims=True)
        g2 = jnp.sum(jnp.where(glane == 2, gm, 0.0), axis=1, keepdims=True)
        comb = g0 * cmp_ref[0, qsl, :] + g1 * out_sel + g2 * out_win
        out_ref[0, qsl, :] = comb.astype(BF)


def _oproj_kernel(o_ref, wot_ref, out_ref):
    acc = jnp.zeros((TS, D), jnp.float32)
    for h in range(H):
        acc = acc + jnp.dot(o_ref[h], wot_ref[h],
                            preferred_element_type=jnp.float32)
    out_ref[...] = acc


def kernel(x, start_pos, freqs_cis, Wq, Wk, Wv, Wo, Wg):
    x2 = x.reshape(S, D)
    wf = jnp.concatenate([Wq, Wk, Wv], axis=0)
    cos = freqs_cis[..., 0]
    sin = freqs_cis[..., 1]
    cosi = jnp.repeat(cos, 2, axis=1)
    sins = jnp.stack([-sin, sin], axis=-1).reshape(S, DH)
    pmat = jnp.zeros((DH, DH), jnp.float32)
    idx = np.arange(0, DH, 2)
    pmat = pmat.at[idx + 1, idx].set(1.0).at[idx, idx + 1].set(1.0)

    qh, kg, vg, qb, kb, vb = pl.pallas_call(
        _proj_kernel,
        grid=(S // TS,),
        in_specs=[
            pl.BlockSpec((TS, D), lambda i: (i, 0)),
            pl.BlockSpec(((H + 2 * G) * DH, D), lambda i: (0, 0)),
            pl.BlockSpec((TS, DH), lambda i: (i, 0)),
            pl.BlockSpec((TS, DH), lambda i: (i, 0)),
            pl.BlockSpec((DH, DH), lambda i: (0, 0)),
        ],
        out_specs=[
            pl.BlockSpec((H, TS, DH), lambda i: (0, i, 0)),
            pl.BlockSpec((G, TS, DH), lambda i: (0, i, 0)),
            pl.BlockSpec((G, TS, DH), lambda i: (0, i, 0)),
            pl.BlockSpec((H, TS, DH), lambda i: (0, i, 0)),
            pl.BlockSpec((G, TS, DH), lambda i: (0, i, 0)),
            pl.BlockSpec((G, TS, DH), lambda i: (0, i, 0)),
        ],
        out_shape=[
            jax.ShapeDtypeStruct((H, S, DH), jnp.float32),
            jax.ShapeDtypeStruct((G, S, DH), jnp.float32),
            jax.ShapeDtypeStruct((G, S, DH), jnp.float32),
            jax.ShapeDtypeStruct((H, S, DH), BF),
            jax.ShapeDtypeStruct((G, S, DH), BF),
            jax.ShapeDtypeStruct((G, S, DH), BF),
        ],
        compiler_params=pltpu.CompilerParams(
            dimension_semantics=("parallel",)),
    )(x2, wf, cosi, sins, pmat)

    qg = qh.reshape(G, HG, S, DH)

    cidx = np.arange(CP)
    tidx = np.arange(S)
    win_np = ((tidx[None, :] >= STRIDE * cidx[:, None])
              & (tidx[None, :] < STRIDE * cidx[:, None] + L)
              & (cidx[:, None] < C)).astype(np.float32) / L
    wint = jnp.asarray(win_np)
    bstart = np.arange(NB) * LP
    ov_np = ((STRIDE * cidx[:, None] <= bstart[None, :] + LP - 1)
             & (STRIDE * cidx[:, None] + L - 1 >= bstart[None, :])
             & (cidx[:, None] < C)).astype(np.float32)
    ov = jnp.asarray(ov_np)

    out_cmp, blk_sel = pl.pallas_call(
        _cmp_kernel,
        grid=(G,),
        in_specs=[
            pl.BlockSpec((1, HG, S, DH), lambda g: (g, 0, 0, 0)),
            pl.BlockSpec((1, S, DH), lambda g: (g, 0, 0)),
            pl.BlockSpec((1, S, DH), lambda g: (g, 0, 0)),
            pl.BlockSpec((CP, S), lambda g: (0, 0)),
            pl.BlockSpec((CP, NB), lambda g: (0, 0)),
        ],
        out_specs=[
            pl.BlockSpec((1, HG, S, DH), lambda g: (g, 0, 0, 0)),
            pl.BlockSpec((1, S, NB), lambda g: (g, 0, 0)),
        ],
        out_shape=[
            jax.ShapeDtypeStruct((G, HG, S, DH), jnp.float32),
            jax.ShapeDtypeStruct((G, S, NB), jnp.float32),
        ],
        compiler_params=pltpu.CompilerParams(
            dimension_semantics=("parallel",)),
    )(qg, kg, vg, wint, ov)

    cmp_h = out_cmp.reshape(H, S, DH)
    wg_pad = jnp.zeros((DH, CP), BF).at[:, :3].set(Wg.astype(BF))
    e4_np = (np.arange(KT)[None, :] // LP == np.arange(BPT)[:, None])
    e4 = jnp.asarray(e4_np.astype(np.float32))

    out_h = pl.pallas_call(
        _flash_kernel,
        grid=(H,),
        in_specs=[
            pl.BlockSpec((1, S, DH), lambda h: (h, 0, 0)),
            pl.BlockSpec((1, S, DH), lambda h: (h // HG, 0, 0)),
            pl.BlockSpec((1, S, DH), lambda h: (h // HG, 0, 0)),
            pl.BlockSpec((1, S, NB), lambda h: (h // HG, 0, 0)),
            pl.BlockSpec((1, S, DH), lambda h: (h, 0, 0)),
            pl.BlockSpec((DH, CP), lambda h: (0, 0)),
            pl.BlockSpec((BPT, KT), lambda h: (0, 0)),
        ],
        out_specs=pl.BlockSpec((1, S, DH), lambda h: (h, 0, 0)),
        out_shape=jax.ShapeDtypeStruct((H, S, DH), BF),
        compiler_params=pltpu.CompilerParams(
            dimension_semantics=("parallel",)),
    )(qb, kb, vb, blk_sel, cmp_h, wg_pad, e4)

    wot = Wo.T.reshape(H, DH, D).astype(BF)
    out = pl.pallas_call(
        _oproj_kernel,
        grid=(S // TS,),
        in_specs=[
            pl.BlockSpec((H, TS, DH), lambda i: (0, i, 0)),
            pl.BlockSpec((H, DH, D), lambda i: (0, 0, 0)),
        ],
        out_specs=pl.BlockSpec((TS, D), lambda i: (i, 0)),
        out_shape=jax.ShapeDtypeStruct((S, D), jnp.float32),
        compiler_params=pltpu.CompilerParams(
            dimension_semantics=("parallel",)),
    )(out_h, wot)
    return out.reshape(B, S, D)


# speedup vs baseline: 5.3457x; 1.0667x over previous
import jax
import jax.numpy as jnp
import numpy as np
from jax.experimental import pallas as pl
from jax.experimental.pallas import tpu as pltpu

B, S, D, H, G, DH = 1, 2048, 1024, 16, 4, 64
HG = H // G
L, STRIDE, LP, NSEL, W = 32, 16, 64, 8, 512
C = (S - L) // STRIDE + 1
CP = 128
NB = S // LP
SCALE = 1.0 / np.sqrt(DH)
NEG = -1e30

TS = 256
TQ = 256
KT = 256
BPT = KT // LP
RQ = HG * TQ
BF = jnp.bfloat16


def _proj_kernel(x_ref, wf_ref, cosi_ref, sins_ref, p_ref, q_ref, k_ref,
                 v_ref):
    xw = jax.lax.dot_general(x_ref[...], wf_ref[...],
                             (((1,), (1,)), ((), ())),
                             preferred_element_type=jnp.float32)
    cosi = cosi_ref[...]
    sins = sins_ref[...]
    pm = p_ref[...]
    for h in range(H):
        u = xw[:, h * DH:(h + 1) * DH]
        q_ref[h] = u * cosi + (u @ pm) * sins
    for g in range(G):
        u = xw[:, H * DH + g * DH: H * DH + (g + 1) * DH]
        k_ref[g] = u * cosi + (u @ pm) * sins
        v_ref[g] = xw[:, (H + G) * DH + g * DH: (H + G) * DH + (g + 1) * DH]


def _attn_kernel(q_ref, k_ref, v_ref, wint_ref, ov_ref, wg_ref, e4_ref,
                 out_ref, cmp_scr, kb_scr, vb_scr):
    kf = k_ref[0]
    vf = v_ref[0]
    kc = jnp.dot(wint_ref[...], kf, preferred_element_type=jnp.float32)
    vc = jnp.dot(wint_ref[...], vf, preferred_element_type=jnp.float32)
    s_iota = jax.lax.broadcasted_iota(jnp.int32, (S, CP), 0)
    c_iota = jax.lax.broadcasted_iota(jnp.int32, (S, CP), 1)
    cmask = (STRIDE * c_iota + L - 1) <= s_iota
    cmask_f = cmask.astype(jnp.float32)
    imp = jnp.zeros((S, NB), jnp.float32)
    for h in range(HG):
        sc = jax.lax.dot_general(q_ref[h], kc, (((1,), (1,)), ((), ())),
                                 preferred_element_type=jnp.float32) * SCALE
        sc = jnp.where(cmask, sc, NEG)
        mx = jnp.max(sc, axis=1, keepdims=True)
        e = jnp.exp(sc - mx) * cmask_f
        p = e / jnp.maximum(jnp.sum(e, axis=1, keepdims=True), 1e-30)
        cmp_scr[h] = jnp.dot(p, vc, preferred_element_type=jnp.float32)
        imp = imp + jnp.dot(p, ov_ref[...], preferred_element_type=jnp.float32)
    sj = jax.lax.broadcasted_iota(jnp.int32, (S, NB), 0)
    bj = jax.lax.broadcasted_iota(jnp.int32, (S, NB), 1)
    forced = (bj == sj // LP) | (bj == 0)
    quota = jnp.where(sj[:, :1] // LP == 0, NSEL - 1, NSEL - 2)
    selected = forced.astype(jnp.float32)
    work = jnp.where(forced, -1.0, imp)
    for i in range(NSEL - 1):
        mx = jnp.max(work, axis=1, keepdims=True)
        pick = (work == mx) & (i < quota)
        selected = jnp.where(pick, 1.0, selected)
        work = jnp.where(pick, -1.0, work)

    kb_scr[...] = (kf * SCALE).astype(BF)
    vb_scr[...] = vf.astype(BF)
    lane = jax.lax.broadcasted_iota(jnp.int32, (RQ, KT), 1)
    rowq = jax.lax.broadcasted_iota(jnp.int32, (RQ, KT), 0) % TQ
    dd = lane - rowq
    causal = dd <= 0
    edge = dd > 0
    edge_f = edge.astype(jnp.float32)
    glane = jax.lax.broadcasted_iota(jnp.int32, (RQ, CP), 1)
    e4 = e4_ref[...]
    wg = wg_ref[...]

    for qt in range(S // TQ):
        qsl = slice(qt * TQ, (qt + 1) * TQ)
        q4 = jnp.concatenate([q_ref[h, qsl, :] for h in range(HG)], axis=0)
        q4b = q4.astype(BF)
        bs = selected[qsl]
        bs4 = jnp.concatenate([bs] * HG, axis=0)
        m_s = jnp.full((RQ, 1), NEG)
        l_s = jnp.zeros((RQ, 1), jnp.float32)
        a_s = jnp.zeros((RQ, DH), jnp.float32)
        m_w = jnp.full((RQ, 1), NEG)
        l_w = jnp.zeros((RQ, 1), jnp.float32)
        a_w = jnp.zeros((RQ, DH), jnp.float32)
        for kt in range(qt + 1):
            ksl = slice(kt * KT, (kt + 1) * KT)
            kblk = kb_scr[ksl, :]
            vblk = vb_scr[ksl, :]
            sf = jax.lax.dot_general(q4b, kblk, (((1,), (1,)), ((), ())),
                                     preferred_element_type=jnp.float32)
            if kt == qt:
                sf = jnp.where(causal, sf, NEG)
            smf = jnp.dot(bs4[:, kt * BPT:(kt + 1) * BPT], e4,
                          preferred_element_type=jnp.float32)
            sm = smf > 0.5
            mn = jnp.maximum(m_s, jnp.max(jnp.where(sm, sf, NEG), axis=1,
                                          keepdims=True))
            p = jnp.exp(sf - mn) * smf
            alpha = jnp.exp(m_s - mn)
            l_s = l_s * alpha + jnp.sum(p, axis=1, keepdims=True)
            a_s = a_s * alpha + jnp.dot(p.astype(BF), vblk,
                                        preferred_element_type=jnp.float32)
            m_s = mn
            if kt >= qt - 2:
                if kt == qt - 2:
                    mnw = jnp.maximum(m_w, jnp.max(jnp.where(edge, sf, NEG),
                                                   axis=1, keepdims=True))
                    pw = jnp.exp(sf - mnw) * edge_f
                else:
                    mnw = jnp.maximum(m_w, jnp.max(sf, axis=1, keepdims=True))
                    pw = jnp.exp(sf - mnw)
                aw = jnp.exp(m_w - mnw)
                l_w = l_w * aw + jnp.sum(pw, axis=1, keepdims=True)
                a_w = a_w * aw + jnp.dot(pw.astype(BF), vblk,
                                         preferred_element_type=jnp.float32)
                m_w = mnw
        out_sel = a_s / jnp.maximum(l_s, 1e-30)
        out_win = a_w / jnp.maximum(l_w, 1e-30)
        gm = jax.nn.sigmoid(jax.lax.dot_general(
            q4b, wg, (((1,), (0,)), ((), ())),
            preferred_element_type=jnp.float32))
        g0 = jnp.sum(jnp.where(glane == 0, gm, 0.0), axis=1, keepdims=True)
        g1 = jnp.sum(jnp.where(glane == 1, gm, 0.0), axis=1, keepdims=True)
        g2 = jnp.sum(jnp.where(glane == 2, gm, 0.0), axis=1, keepdims=True)
        cmp4 = jnp.concatenate([cmp_scr[h, qsl, :] for h in range(HG)],
                               axis=0)
        comb = g0 * cmp4 + g1 * out_sel + g2 * out_win
        for h in range(HG):
            out_ref[qsl, h * DH:(h + 1) * DH] = (
                comb[h * TQ:(h + 1) * TQ].astype(BF))


def _oproj_kernel(o_ref, wo_ref, out_ref):
    out_ref[...] = jax.lax.dot_general(o_ref[...], wo_ref[...],
                                       (((1,), (1,)), ((), ())),
                                       preferred_element_type=jnp.float32)


def kernel(x, start_pos, freqs_cis, Wq, Wk, Wv, Wo, Wg):
    x2 = x.reshape(S, D)
    wf = jnp.concatenate([Wq, Wk, Wv], axis=0)
    cos = freqs_cis[..., 0]
    sin = freqs_cis[..., 1]
    cosi = jnp.repeat(cos, 2, axis=1)
    sins = jnp.stack([-sin, sin], axis=-1).reshape(S, DH)
    pmat = jnp.zeros((DH, DH), jnp.float32)
    idx = np.arange(0, DH, 2)
    pmat = pmat.at[idx + 1, idx].set(1.0).at[idx, idx + 1].set(1.0)

    qh, kg, vg = pl.pallas_call(
        _proj_kernel,
        grid=(S // TS,),
        in_specs=[
            pl.BlockSpec((TS, D), lambda i: (i, 0)),
            pl.BlockSpec(((H + 2 * G) * DH, D), lambda i: (0, 0)),
            pl.BlockSpec((TS, DH), lambda i: (i, 0)),
            pl.BlockSpec((TS, DH), lambda i: (i, 0)),
            pl.BlockSpec((DH, DH), lambda i: (0, 0)),
        ],
        out_specs=[
            pl.BlockSpec((H, TS, DH), lambda i: (0, i, 0)),
            pl.BlockSpec((G, TS, DH), lambda i: (0, i, 0)),
            pl.BlockSpec((G, TS, DH), lambda i: (0, i, 0)),
        ],
        out_shape=[
            jax.ShapeDtypeStruct((H, S, DH), jnp.float32),
            jax.ShapeDtypeStruct((G, S, DH), jnp.float32),
            jax.ShapeDtypeStruct((G, S, DH), jnp.float32),
        ],
        compiler_params=pltpu.CompilerParams(
            dimension_semantics=("parallel",)),
    )(x2, wf, cosi, sins, pmat)

    cidx = np.arange(CP)
    tidx = np.arange(S)
    win_np = ((tidx[None, :] >= STRIDE * cidx[:, None])
              & (tidx[None, :] < STRIDE * cidx[:, None] + L)
              & (cidx[:, None] < C)).astype(np.float32) / L
    wint = jnp.asarray(win_np)
    bstart = np.arange(NB) * LP
    ov_np = ((STRIDE * cidx[:, None] <= bstart[None, :] + LP - 1)
             & (STRIDE * cidx[:, None] + L - 1 >= bstart[None, :])
             & (cidx[:, None] < C)).astype(np.float32)
    ov = jnp.asarray(ov_np)
    wg_pad = jnp.zeros((DH, CP), BF).at[:, :3].set(Wg.astype(BF))
    e4_np = (np.arange(KT)[None, :] // LP == np.arange(BPT)[:, None])
    e4 = jnp.asarray(e4_np.astype(np.float32))

    out_h = pl.pallas_call(
        _attn_kernel,
        grid=(G,),
        in_specs=[
            pl.BlockSpec((HG, S, DH), lambda g: (g, 0, 0)),
            pl.BlockSpec((1, S, DH), lambda g: (g, 0, 0)),
            pl.BlockSpec((1, S, DH), lambda g: (g, 0, 0)),
            pl.BlockSpec((CP, S), lambda g: (0, 0)),
            pl.BlockSpec((CP, NB), lambda g: (0, 0)),
            pl.BlockSpec((DH, CP), lambda g: (0, 0)),
            pl.BlockSpec((BPT, KT), lambda g: (0, 0)),
        ],
        out_specs=pl.BlockSpec((S, HG * DH), lambda g: (0, g)),
        out_shape=jax.ShapeDtypeStruct((S, H * DH), BF),
        scratch_shapes=[
            pltpu.VMEM((HG, S, DH), jnp.float32),
            pltpu.VMEM((S, DH), BF),
            pltpu.VMEM((S, DH), BF),
        ],
        compiler_params=pltpu.CompilerParams(
            dimension_semantics=("parallel",)),
    )(qh, kg, vg, wint, ov, wg_pad, e4)

    wob = Wo.astype(BF)
    out = pl.pallas_call(
        _oproj_kernel,
        grid=(S // TS,),
        in_specs=[
            pl.BlockSpec((TS, H * DH), lambda i: (i, 0)),
            pl.BlockSpec((D, H * DH), lambda i: (0, 0)),
        ],
        out_specs=pl.BlockSpec((TS, D), lambda i: (i, 0)),
        out_shape=jax.ShapeDtypeStruct((S, D), jnp.float32),
        compiler_params=pltpu.CompilerParams(
            dimension_semantics=("parallel",)),
    )(out_h, wob)
    return out.reshape(B, S, D)
